# Initial kernel scaffold; baseline (speedup 1.0000x reference)
import jax
import jax.numpy as jnp
from jax.experimental import pallas as pl


def kernel(all_rois, gt_boxes):
    raise NotImplementedError("write your pallas kernel here")


# speedup vs baseline: 21.1545x; 21.1545x over previous
import functools

import jax
import jax.numpy as jnp
from jax import lax
from jax.experimental import pallas as pl
from jax.experimental.pallas import tpu as pltpu
from jax.experimental.pallas import tpu_sc as plsc

_ROWS = 160
_LANES = 128
_N2P = _ROWS * _LANES
_NCOMP = 18
_FG_THRESH = 0.5


def _tc_body(K, gt_ref, x1_ref, y1_ref, x2_ref, y2_ref, pay_ref, dest_ref):
    x1 = x1_ref[0]
    y1 = y1_ref[0]
    x2 = x2_ref[0]
    y2 = y2_ref[0]
    aw = x2 - x1 + 1.0
    ah = y2 - y1 + 1.0
    an_area = aw * ah
    an_zero = (aw == 1.0) & (ah == 1.0)

    best = None
    for k in range(K):
        gx1 = gt_ref[0, k, 0]
        gy1 = gt_ref[0, k, 1]
        gx2 = gt_ref[0, k, 2]
        gy2 = gt_ref[0, k, 3]
        glab = gt_ref[0, k, 4]
        gw = gx2 - gx1 + 1.0
        gh = gy2 - gy1 + 1.0
        garea = gw * gh
        gzero = (gw == 1.0) & (gh == 1.0)
        iw = jnp.maximum(jnp.minimum(x2, gx2) - jnp.maximum(x1, gx1) + 1.0, 0.0)
        ih = jnp.maximum(jnp.minimum(y2, gy2) - jnp.maximum(y1, gy1) + 1.0, 0.0)
        inter = iw * ih
        ov = inter / (an_area + garea - inter)
        ov = jnp.where(gzero, 0.0, ov)
        ov = jnp.where(an_zero, -1.0, ov)
        if k == 0:
            best = ov
            blab = jnp.zeros_like(ov) + glab
            bx1 = jnp.zeros_like(ov) + gx1
            by1 = jnp.zeros_like(ov) + gy1
            bx2 = jnp.zeros_like(ov) + gx2
            by2 = jnp.zeros_like(ov) + gy2
        else:
            upd = ov > best
            best = jnp.where(upd, ov, best)
            blab = jnp.where(upd, glab, blab)
            bx1 = jnp.where(upd, gx1, bx1)
            by1 = jnp.where(upd, gy1, by1)
            bx2 = jnp.where(upd, gx2, bx2)
            by2 = jnp.where(upd, gy2, by2)

    fg = best >= _FG_THRESH
    label = jnp.where(fg, blab, 0.0)

    ecx = x1 + 0.5 * aw
    ecy = y1 + 0.5 * ah
    gw_v = bx2 - bx1 + 1.0
    gh_v = by2 - by1 + 1.0
    gcx = bx1 + 0.5 * gw_v
    gcy = by1 + 0.5 * gh_v
    t0 = ((gcx - ecx) / aw) / 0.1
    t1 = ((gcy - ecy) / ah) / 0.1
    t2 = jnp.log(gw_v / aw) / 0.2
    t3 = jnp.log(gh_v / ah) / 0.2
    pos = label > 0.0
    zero = jnp.zeros_like(best)
    wt = jnp.where(pos, 1.0, 0.0)

    bidx = jnp.zeros_like(best) + pl.program_id(0).astype(jnp.float32)
    pay_ref[0, 0] = bidx
    pay_ref[0, 1] = x1
    pay_ref[0, 2] = y1
    pay_ref[0, 3] = x2
    pay_ref[0, 4] = y2
    pay_ref[0, 5] = label
    pay_ref[0, 6] = jnp.where(pos, t0, zero)
    pay_ref[0, 7] = jnp.where(pos, t1, zero)
    pay_ref[0, 8] = jnp.where(pos, t2, zero)
    pay_ref[0, 9] = jnp.where(pos, t3, zero)
    for c in range(10, 14):
        pay_ref[0, c] = wt
    for c in range(14, 18):
        pay_ref[0, c] = wt

    fgf = fg.astype(jnp.float32)
    r0 = lax.broadcasted_iota(jnp.int32, (_LANES, _LANES), 0)
    c0 = lax.broadcasted_iota(jnp.int32, (_LANES, _LANES), 1)
    upper = (r0 <= c0).astype(jnp.float32)
    incl = jnp.dot(fgf, upper, preferred_element_type=jnp.float32)
    rowsum = incl[:, _LANES - 1 :]
    rr = lax.broadcasted_iota(jnp.int32, (_ROWS, _ROWS), 0)
    cc = lax.broadcasted_iota(jnp.int32, (_ROWS, _ROWS), 1)
    strict_lower = (cc < rr).astype(jnp.float32)
    offs = jnp.dot(strict_lower, rowsum, preferred_element_type=jnp.float32)
    pf_excl = incl + offs - fgf
    nfg = jnp.sum(fgf)
    lin = (
        lax.broadcasted_iota(jnp.int32, (_ROWS, _LANES), 0) * _LANES
        + lax.broadcasted_iota(jnp.int32, (_ROWS, _LANES), 1)
    ).astype(jnp.float32)
    destf = jnp.where(fg, pf_excl, nfg + lin - pf_excl)
    dest_ref[0] = destf.astype(jnp.int32)


def _sc_body(ntask, nchunk, pay_hbm, dest_hbm, out_hbm, dest_v, in_v, out_v):
    info = plsc.get_sparse_core_info()
    nworker = info.num_cores * info.num_subcores
    wid = lax.axis_index("s") * info.num_cores + lax.axis_index("c")
    nslot = (ntask + nworker - 1) // nworker
    for s in range(nslot):
        t = wid + s * nworker

        @pl.when(t < ntask)
        def _():
            b = t // _NCOMP
            c = t % _NCOMP
            pltpu.sync_copy(dest_hbm.at[b], dest_v)
            pltpu.sync_copy(pay_hbm.at[b, c], in_v)

            def body(i, carry):
                off = i * 16
                d = dest_v[pl.ds(off, 16)]
                v = in_v[pl.ds(off, 16)]
                plsc.store_scatter(out_v, [d], v)
                return carry

            lax.fori_loop(0, nchunk, body, 0)
            pltpu.sync_copy(out_v, out_hbm.at[b, c])


def kernel(all_rois, gt_boxes):
    B, N, _ = all_rois.shape
    K = gt_boxes.shape[1]
    N2 = N + K
    pad = _N2P - N2
    z = jnp.zeros((B, pad), dtype=jnp.float32)
    x1 = jnp.concatenate([all_rois[:, :, 1], gt_boxes[:, :, 0], z], axis=1)
    y1 = jnp.concatenate([all_rois[:, :, 2], gt_boxes[:, :, 1], z], axis=1)
    x2 = jnp.concatenate([all_rois[:, :, 3], gt_boxes[:, :, 2], z], axis=1)
    y2 = jnp.concatenate([all_rois[:, :, 4], gt_boxes[:, :, 3], z], axis=1)
    shape3 = (B, _ROWS, _LANES)
    x1, y1, x2, y2 = (a.reshape(shape3) for a in (x1, y1, x2, y2))

    pay, dest = pl.pallas_call(
        functools.partial(_tc_body, K),
        grid=(B,),
        in_specs=[
            pl.BlockSpec((1, K, 5), lambda b: (b, 0, 0), memory_space=pltpu.SMEM),
            pl.BlockSpec((1, _ROWS, _LANES), lambda b: (b, 0, 0)),
            pl.BlockSpec((1, _ROWS, _LANES), lambda b: (b, 0, 0)),
            pl.BlockSpec((1, _ROWS, _LANES), lambda b: (b, 0, 0)),
            pl.BlockSpec((1, _ROWS, _LANES), lambda b: (b, 0, 0)),
        ],
        out_specs=[
            pl.BlockSpec((1, _NCOMP, _ROWS, _LANES), lambda b: (b, 0, 0, 0)),
            pl.BlockSpec((1, _ROWS, _LANES), lambda b: (b, 0, 0)),
        ],
        out_shape=[
            jax.ShapeDtypeStruct((B, _NCOMP, _ROWS, _LANES), jnp.float32),
            jax.ShapeDtypeStruct((B, _ROWS, _LANES), jnp.int32),
        ],
    )(gt_boxes, x1, y1, x2, y2)

    pay = pay.reshape(B, _NCOMP, _N2P)
    dest = dest.reshape(B, _N2P)

    mesh = plsc.VectorSubcoreMesh(core_axis_name="c", subcore_axis_name="s")
    out = pl.kernel(
        functools.partial(_sc_body, B * _NCOMP, _N2P // 16),
        out_type=jax.ShapeDtypeStruct((B, _NCOMP, _N2P), jnp.float32),
        mesh=mesh,
        compiler_params=pltpu.CompilerParams(
            use_tc_tiling_on_sc=False, needs_layout_passes=False
        ),
        scratch_types=[
            pltpu.VMEM((_N2P,), jnp.int32),
            pltpu.VMEM((_N2P,), jnp.float32),
            pltpu.VMEM((_N2P,), jnp.float32),
        ],
    )(pay, dest)

    o = out[:, :, :N2]
    rois_batch = jnp.moveaxis(o[:, 0:5], 1, 2)
    labels_batch = o[:, 5]
    bbox_targets = jnp.moveaxis(o[:, 6:10], 1, 2)
    bbox_inside_weights = jnp.moveaxis(o[:, 10:14], 1, 2)
    bbox_outside_weights = jnp.moveaxis(o[:, 14:18], 1, 2)
    return (
        rois_batch,
        labels_batch,
        bbox_targets,
        bbox_inside_weights,
        bbox_outside_weights,
    )


# speedup vs baseline: 23.2953x; 1.1012x over previous
import functools

import jax
import jax.numpy as jnp
from jax import lax
from jax.experimental import pallas as pl
from jax.experimental.pallas import tpu as pltpu
from jax.experimental.pallas import tpu_sc as plsc

_ROWS = 160
_LANES = 128
_N2P = _ROWS * _LANES
_NCOMP = 18
_FG_THRESH = 0.5


def _tc_body(K, gt_ref, x1_ref, y1_ref, x2_ref, y2_ref, pay_ref, dest_ref):
    x1 = x1_ref[0]
    y1 = y1_ref[0]
    x2 = x2_ref[0]
    y2 = y2_ref[0]
    aw = x2 - x1 + 1.0
    ah = y2 - y1 + 1.0
    an_area = aw * ah
    an_zero = (aw == 1.0) & (ah == 1.0)

    best = None
    for k in range(K):
        gx1 = gt_ref[0, k, 0]
        gy1 = gt_ref[0, k, 1]
        gx2 = gt_ref[0, k, 2]
        gy2 = gt_ref[0, k, 3]
        glab = gt_ref[0, k, 4]
        gw = gx2 - gx1 + 1.0
        gh = gy2 - gy1 + 1.0
        garea = gw * gh
        gzero = (gw == 1.0) & (gh == 1.0)
        iw = jnp.maximum(jnp.minimum(x2, gx2) - jnp.maximum(x1, gx1) + 1.0, 0.0)
        ih = jnp.maximum(jnp.minimum(y2, gy2) - jnp.maximum(y1, gy1) + 1.0, 0.0)
        inter = iw * ih
        ov = inter / (an_area + garea - inter)
        ov = jnp.where(gzero, 0.0, ov)
        ov = jnp.where(an_zero, -1.0, ov)
        if k == 0:
            best = ov
            blab = jnp.zeros_like(ov) + glab
            bx1 = jnp.zeros_like(ov) + gx1
            by1 = jnp.zeros_like(ov) + gy1
            bx2 = jnp.zeros_like(ov) + gx2
            by2 = jnp.zeros_like(ov) + gy2
        else:
            upd = ov > best
            best = jnp.where(upd, ov, best)
            blab = jnp.where(upd, glab, blab)
            bx1 = jnp.where(upd, gx1, bx1)
            by1 = jnp.where(upd, gy1, by1)
            bx2 = jnp.where(upd, gx2, bx2)
            by2 = jnp.where(upd, gy2, by2)

    fg = best >= _FG_THRESH
    label = jnp.where(fg, blab, 0.0)

    ecx = x1 + 0.5 * aw
    ecy = y1 + 0.5 * ah
    gw_v = bx2 - bx1 + 1.0
    gh_v = by2 - by1 + 1.0
    gcx = bx1 + 0.5 * gw_v
    gcy = by1 + 0.5 * gh_v
    t0 = ((gcx - ecx) / aw) / 0.1
    t1 = ((gcy - ecy) / ah) / 0.1
    t2 = jnp.log(gw_v / aw) / 0.2
    t3 = jnp.log(gh_v / ah) / 0.2
    pos = label > 0.0
    zero = jnp.zeros_like(best)
    wt = jnp.where(pos, 1.0, 0.0)

    bidx = jnp.zeros_like(best) + pl.program_id(0).astype(jnp.float32)
    pay_ref[0, 0] = bidx
    pay_ref[0, 1] = x1
    pay_ref[0, 2] = y1
    pay_ref[0, 3] = x2
    pay_ref[0, 4] = y2
    pay_ref[0, 5] = label
    pay_ref[0, 6] = jnp.where(pos, t0, zero)
    pay_ref[0, 7] = jnp.where(pos, t1, zero)
    pay_ref[0, 8] = jnp.where(pos, t2, zero)
    pay_ref[0, 9] = jnp.where(pos, t3, zero)
    for c in range(10, 14):
        pay_ref[0, c] = wt
    for c in range(14, 18):
        pay_ref[0, c] = wt

    fgf = fg.astype(jnp.float32)
    r0 = lax.broadcasted_iota(jnp.int32, (_LANES, _LANES), 0)
    c0 = lax.broadcasted_iota(jnp.int32, (_LANES, _LANES), 1)
    upper = (r0 <= c0).astype(jnp.float32)
    incl = jnp.dot(fgf, upper, preferred_element_type=jnp.float32)
    rowsum = incl[:, _LANES - 1 :]
    rr = lax.broadcasted_iota(jnp.int32, (_ROWS, _ROWS), 0)
    cc = lax.broadcasted_iota(jnp.int32, (_ROWS, _ROWS), 1)
    strict_lower = (cc < rr).astype(jnp.float32)
    offs = jnp.dot(strict_lower, rowsum, preferred_element_type=jnp.float32)
    pf_excl = incl + offs - fgf
    nfg = jnp.sum(fgf)
    lin = (
        lax.broadcasted_iota(jnp.int32, (_ROWS, _LANES), 0) * _LANES
        + lax.broadcasted_iota(jnp.int32, (_ROWS, _LANES), 1)
    ).astype(jnp.float32)
    destf = jnp.where(fg, pf_excl, nfg + lin - pf_excl)
    dest_ref[0] = destf.astype(jnp.int32)


_UNROLL = 8


def _sc_body(B, nchunk, pay_hbm, dest_hbm, out_hbm, dest_v, in_v, out_v):
    info = plsc.get_sparse_core_info()
    nworker = info.num_cores * info.num_subcores
    wid = lax.axis_index("s") * info.num_cores + lax.axis_index("c")
    wpb = nworker // B
    b = wid // wpb
    lane = wid % wpb
    pltpu.sync_copy(dest_hbm.at[b], dest_v)
    for s in range((_NCOMP + wpb - 1) // wpb):
        c = lane + s * wpb

        @pl.when(c < _NCOMP)
        def _():
            pltpu.sync_copy(pay_hbm.at[b, c], in_v)

            def body(i, carry):
                off = i * (16 * _UNROLL)
                for u in range(_UNROLL):
                    d = dest_v[pl.ds(off + u * 16, 16)]
                    v = in_v[pl.ds(off + u * 16, 16)]
                    plsc.store_scatter(out_v, [d], v)
                return carry

            lax.fori_loop(0, nchunk // _UNROLL, body, 0)
            pltpu.sync_copy(out_v, out_hbm.at[b, c])


def kernel(all_rois, gt_boxes):
    B, N, _ = all_rois.shape
    K = gt_boxes.shape[1]
    N2 = N + K
    pad = _N2P - N2
    z = jnp.zeros((B, pad), dtype=jnp.float32)
    x1 = jnp.concatenate([all_rois[:, :, 1], gt_boxes[:, :, 0], z], axis=1)
    y1 = jnp.concatenate([all_rois[:, :, 2], gt_boxes[:, :, 1], z], axis=1)
    x2 = jnp.concatenate([all_rois[:, :, 3], gt_boxes[:, :, 2], z], axis=1)
    y2 = jnp.concatenate([all_rois[:, :, 4], gt_boxes[:, :, 3], z], axis=1)
    shape3 = (B, _ROWS, _LANES)
    x1, y1, x2, y2 = (a.reshape(shape3) for a in (x1, y1, x2, y2))

    pay, dest = pl.pallas_call(
        functools.partial(_tc_body, K),
        grid=(B,),
        in_specs=[
            pl.BlockSpec((1, K, 5), lambda b: (b, 0, 0), memory_space=pltpu.SMEM),
            pl.BlockSpec((1, _ROWS, _LANES), lambda b: (b, 0, 0)),
            pl.BlockSpec((1, _ROWS, _LANES), lambda b: (b, 0, 0)),
            pl.BlockSpec((1, _ROWS, _LANES), lambda b: (b, 0, 0)),
            pl.BlockSpec((1, _ROWS, _LANES), lambda b: (b, 0, 0)),
        ],
        out_specs=[
            pl.BlockSpec((1, _NCOMP, _ROWS, _LANES), lambda b: (b, 0, 0, 0)),
            pl.BlockSpec((1, _ROWS, _LANES), lambda b: (b, 0, 0)),
        ],
        out_shape=[
            jax.ShapeDtypeStruct((B, _NCOMP, _ROWS, _LANES), jnp.float32),
            jax.ShapeDtypeStruct((B, _ROWS, _LANES), jnp.int32),
        ],
    )(gt_boxes, x1, y1, x2, y2)

    pay = pay.reshape(B, _NCOMP, _N2P)
    dest = dest.reshape(B, _N2P)

    mesh = plsc.VectorSubcoreMesh(core_axis_name="c", subcore_axis_name="s")
    out = pl.kernel(
        functools.partial(_sc_body, B, _N2P // 16),
        out_type=jax.ShapeDtypeStruct((B, _NCOMP, _N2P), jnp.float32),
        mesh=mesh,
        compiler_params=pltpu.CompilerParams(
            use_tc_tiling_on_sc=False, needs_layout_passes=False
        ),
        scratch_types=[
            pltpu.VMEM((_N2P,), jnp.int32),
            pltpu.VMEM((_N2P,), jnp.float32),
            pltpu.VMEM((_N2P,), jnp.float32),
        ],
    )(pay, dest)

    o = out[:, :, :N2]
    rois_batch = jnp.moveaxis(o[:, 0:5], 1, 2)
    labels_batch = o[:, 5]
    bbox_targets = jnp.moveaxis(o[:, 6:10], 1, 2)
    bbox_inside_weights = jnp.moveaxis(o[:, 10:14], 1, 2)
    bbox_outside_weights = jnp.moveaxis(o[:, 14:18], 1, 2)
    return (
        rois_batch,
        labels_batch,
        bbox_targets,
        bbox_inside_weights,
        bbox_outside_weights,
    )


# speedup vs baseline: 30.8487x; 1.3242x over previous
import functools

import jax
import jax.numpy as jnp
from jax import lax
from jax.experimental import pallas as pl
from jax.experimental.pallas import tpu as pltpu
from jax.experimental.pallas import tpu_sc as plsc

_ROWS = 160
_LANES = 128
_N2P = _ROWS * _LANES
_NCOMP = 9
_FG_THRESH = 0.5


def _tc_body(K, gt_ref, x1_ref, y1_ref, x2_ref, y2_ref, pay_ref, dest_ref):
    x1 = x1_ref[0]
    y1 = y1_ref[0]
    x2 = x2_ref[0]
    y2 = y2_ref[0]
    aw = x2 - x1 + 1.0
    ah = y2 - y1 + 1.0
    an_area = aw * ah
    an_zero = (aw == 1.0) & (ah == 1.0)

    best = None
    for k in range(K):
        gx1 = gt_ref[0, k, 0]
        gy1 = gt_ref[0, k, 1]
        gx2 = gt_ref[0, k, 2]
        gy2 = gt_ref[0, k, 3]
        glab = gt_ref[0, k, 4]
        gw = gx2 - gx1 + 1.0
        gh = gy2 - gy1 + 1.0
        garea = gw * gh
        gzero = (gw == 1.0) & (gh == 1.0)
        iw = jnp.maximum(jnp.minimum(x2, gx2) - jnp.maximum(x1, gx1) + 1.0, 0.0)
        ih = jnp.maximum(jnp.minimum(y2, gy2) - jnp.maximum(y1, gy1) + 1.0, 0.0)
        inter = iw * ih
        ov = inter / (an_area + garea - inter)
        ov = jnp.where(gzero, 0.0, ov)
        ov = jnp.where(an_zero, -1.0, ov)
        if k == 0:
            best = ov
            blab = jnp.zeros_like(ov) + glab
            bx1 = jnp.zeros_like(ov) + gx1
            by1 = jnp.zeros_like(ov) + gy1
            bx2 = jnp.zeros_like(ov) + gx2
            by2 = jnp.zeros_like(ov) + gy2
        else:
            upd = ov > best
            best = jnp.where(upd, ov, best)
            blab = jnp.where(upd, glab, blab)
            bx1 = jnp.where(upd, gx1, bx1)
            by1 = jnp.where(upd, gy1, by1)
            bx2 = jnp.where(upd, gx2, bx2)
            by2 = jnp.where(upd, gy2, by2)

    fg = best >= _FG_THRESH
    label = jnp.where(fg, blab, 0.0)

    ecx = x1 + 0.5 * aw
    ecy = y1 + 0.5 * ah
    gw_v = bx2 - bx1 + 1.0
    gh_v = by2 - by1 + 1.0
    gcx = bx1 + 0.5 * gw_v
    gcy = by1 + 0.5 * gh_v
    t0 = ((gcx - ecx) / aw) / 0.1
    t1 = ((gcy - ecy) / ah) / 0.1
    t2 = jnp.log(gw_v / aw) / 0.2
    t3 = jnp.log(gh_v / ah) / 0.2
    pos = label > 0.0
    zero = jnp.zeros_like(best)

    pay_ref[0, 0] = x1
    pay_ref[0, 1] = y1
    pay_ref[0, 2] = x2
    pay_ref[0, 3] = y2
    pay_ref[0, 4] = label
    pay_ref[0, 5] = jnp.where(pos, t0, zero)
    pay_ref[0, 6] = jnp.where(pos, t1, zero)
    pay_ref[0, 7] = jnp.where(pos, t2, zero)
    pay_ref[0, 8] = jnp.where(pos, t3, zero)

    fgf = fg.astype(jnp.float32)
    r0 = lax.broadcasted_iota(jnp.int32, (_LANES, _LANES), 0)
    c0 = lax.broadcasted_iota(jnp.int32, (_LANES, _LANES), 1)
    upper = (r0 <= c0).astype(jnp.float32)
    incl = jnp.dot(fgf, upper, preferred_element_type=jnp.float32)
    rowsum = incl[:, _LANES - 1 :]
    rr = lax.broadcasted_iota(jnp.int32, (_ROWS, _ROWS), 0)
    cc = lax.broadcasted_iota(jnp.int32, (_ROWS, _ROWS), 1)
    strict_lower = (cc < rr).astype(jnp.float32)
    offs = jnp.dot(strict_lower, rowsum, preferred_element_type=jnp.float32)
    pf_excl = incl + offs - fgf
    nfg = jnp.sum(fgf)
    lin = (
        lax.broadcasted_iota(jnp.int32, (_ROWS, _LANES), 0) * _LANES
        + lax.broadcasted_iota(jnp.int32, (_ROWS, _LANES), 1)
    ).astype(jnp.float32)
    destf = jnp.where(fg, pf_excl, nfg + lin - pf_excl)
    dest_ref[0] = destf.astype(jnp.int32)


_UNROLL = 8


def _sc_body(B, nchunk, pay_hbm, dest_hbm, out_hbm, dest_v, in_v, out_v):
    info = plsc.get_sparse_core_info()
    nworker = info.num_cores * info.num_subcores
    wid = lax.axis_index("s") * info.num_cores + lax.axis_index("c")
    wpb = nworker // B
    b = wid // wpb
    lane = wid % wpb
    pltpu.sync_copy(dest_hbm.at[b], dest_v)
    for s in range((_NCOMP + wpb - 1) // wpb):
        c = lane + s * wpb

        @pl.when(c < _NCOMP)
        def _():
            pltpu.sync_copy(pay_hbm.at[b, c], in_v)

            def body(i, carry):
                off = i * (16 * _UNROLL)
                for u in range(_UNROLL):
                    d = dest_v[pl.ds(off + u * 16, 16)]
                    v = in_v[pl.ds(off + u * 16, 16)]
                    plsc.store_scatter(out_v, [d], v)
                return carry

            lax.fori_loop(0, nchunk // _UNROLL, body, 0)
            pltpu.sync_copy(out_v, out_hbm.at[b, c])


def kernel(all_rois, gt_boxes):
    B, N, _ = all_rois.shape
    K = gt_boxes.shape[1]
    N2 = N + K
    pad = _N2P - N2
    z = jnp.zeros((B, pad), dtype=jnp.float32)
    x1 = jnp.concatenate([all_rois[:, :, 1], gt_boxes[:, :, 0], z], axis=1)
    y1 = jnp.concatenate([all_rois[:, :, 2], gt_boxes[:, :, 1], z], axis=1)
    x2 = jnp.concatenate([all_rois[:, :, 3], gt_boxes[:, :, 2], z], axis=1)
    y2 = jnp.concatenate([all_rois[:, :, 4], gt_boxes[:, :, 3], z], axis=1)
    shape3 = (B, _ROWS, _LANES)
    x1, y1, x2, y2 = (a.reshape(shape3) for a in (x1, y1, x2, y2))

    pay, dest = pl.pallas_call(
        functools.partial(_tc_body, K),
        grid=(B,),
        in_specs=[
            pl.BlockSpec((1, K, 5), lambda b: (b, 0, 0), memory_space=pltpu.SMEM),
            pl.BlockSpec((1, _ROWS, _LANES), lambda b: (b, 0, 0)),
            pl.BlockSpec((1, _ROWS, _LANES), lambda b: (b, 0, 0)),
            pl.BlockSpec((1, _ROWS, _LANES), lambda b: (b, 0, 0)),
            pl.BlockSpec((1, _ROWS, _LANES), lambda b: (b, 0, 0)),
        ],
        out_specs=[
            pl.BlockSpec((1, _NCOMP, _ROWS, _LANES), lambda b: (b, 0, 0, 0)),
            pl.BlockSpec((1, _ROWS, _LANES), lambda b: (b, 0, 0)),
        ],
        out_shape=[
            jax.ShapeDtypeStruct((B, _NCOMP, _ROWS, _LANES), jnp.float32),
            jax.ShapeDtypeStruct((B, _ROWS, _LANES), jnp.int32),
        ],
    )(gt_boxes, x1, y1, x2, y2)

    pay = pay.reshape(B, _NCOMP, _N2P)
    dest = dest.reshape(B, _N2P)

    mesh = plsc.VectorSubcoreMesh(core_axis_name="c", subcore_axis_name="s")
    out = pl.kernel(
        functools.partial(_sc_body, B, _N2P // 16),
        out_type=jax.ShapeDtypeStruct((B, _NCOMP, _N2P), jnp.float32),
        mesh=mesh,
        compiler_params=pltpu.CompilerParams(
            use_tc_tiling_on_sc=False, needs_layout_passes=False
        ),
        scratch_types=[
            pltpu.VMEM((_N2P,), jnp.int32),
            pltpu.VMEM((_N2P,), jnp.float32),
            pltpu.VMEM((_N2P,), jnp.float32),
        ],
    )(pay, dest)

    o = out[:, :, :N2]
    labels_batch = o[:, 4]
    bcol = jnp.broadcast_to(
        jnp.arange(B, dtype=jnp.float32)[:, None, None], (B, N2, 1)
    )
    rois_batch = jnp.concatenate([bcol, jnp.moveaxis(o[:, 0:4], 1, 2)], axis=2)
    bbox_targets = jnp.moveaxis(o[:, 5:9], 1, 2)
    pos4 = jnp.broadcast_to((labels_batch > 0)[:, :, None], (B, N2, 4))
    bbox_inside_weights = pos4.astype(jnp.float32)
    bbox_outside_weights = pos4.astype(jnp.float32)
    return (
        rois_batch,
        labels_batch,
        bbox_targets,
        bbox_inside_weights,
        bbox_outside_weights,
    )


# speedup vs baseline: 32.3769x; 1.0495x over previous
import functools

import jax
import jax.numpy as jnp
from jax import lax
from jax.experimental import pallas as pl
from jax.experimental.pallas import tpu as pltpu
from jax.experimental.pallas import tpu_sc as plsc

_ROWS = 160
_LANES = 128
_N2P = _ROWS * _LANES
_NCOMP = 9
_FG_THRESH = 0.5


def _tc_body(K, gt_ref, x1_ref, y1_ref, x2_ref, y2_ref, pay_ref, dest_ref):
    x1 = x1_ref[0]
    y1 = y1_ref[0]
    x2 = x2_ref[0]
    y2 = y2_ref[0]
    aw = x2 - x1 + 1.0
    ah = y2 - y1 + 1.0
    an_area = aw * ah
    an_zero = (aw == 1.0) & (ah == 1.0)

    best = None
    for k in range(K):
        gx1 = gt_ref[0, k, 0]
        gy1 = gt_ref[0, k, 1]
        gx2 = gt_ref[0, k, 2]
        gy2 = gt_ref[0, k, 3]
        glab = gt_ref[0, k, 4]
        gw = gx2 - gx1 + 1.0
        gh = gy2 - gy1 + 1.0
        garea = gw * gh
        gzero = (gw == 1.0) & (gh == 1.0)
        iw = jnp.maximum(jnp.minimum(x2, gx2) - jnp.maximum(x1, gx1) + 1.0, 0.0)
        ih = jnp.maximum(jnp.minimum(y2, gy2) - jnp.maximum(y1, gy1) + 1.0, 0.0)
        inter = iw * ih
        ov = inter / (an_area + garea - inter)
        ov = jnp.where(gzero, 0.0, ov)
        if k == 0:
            best = ov
            blab = jnp.zeros_like(ov) + glab
            bx1 = jnp.zeros_like(ov) + gx1
            by1 = jnp.zeros_like(ov) + gy1
            bx2 = jnp.zeros_like(ov) + gx2
            by2 = jnp.zeros_like(ov) + gy2
        else:
            upd = ov > best
            best = jnp.where(upd, ov, best)
            blab = jnp.where(upd, glab, blab)
            bx1 = jnp.where(upd, gx1, bx1)
            by1 = jnp.where(upd, gy1, by1)
            bx2 = jnp.where(upd, gx2, bx2)
            by2 = jnp.where(upd, gy2, by2)

    best = jnp.where(an_zero, -1.0, best)
    fg = best >= _FG_THRESH
    label = jnp.where(fg, blab, 0.0)

    ecx = x1 + 0.5 * aw
    ecy = y1 + 0.5 * ah
    gw_v = bx2 - bx1 + 1.0
    gh_v = by2 - by1 + 1.0
    gcx = bx1 + 0.5 * gw_v
    gcy = by1 + 0.5 * gh_v
    t0 = ((gcx - ecx) / aw) / 0.1
    t1 = ((gcy - ecy) / ah) / 0.1
    t2 = jnp.log(gw_v / aw) / 0.2
    t3 = jnp.log(gh_v / ah) / 0.2
    pos = label > 0.0
    zero = jnp.zeros_like(best)

    pay_ref[0, 0] = x1
    pay_ref[0, 1] = y1
    pay_ref[0, 2] = x2
    pay_ref[0, 3] = y2
    pay_ref[0, 4] = label
    pay_ref[0, 5] = jnp.where(pos, t0, zero)
    pay_ref[0, 6] = jnp.where(pos, t1, zero)
    pay_ref[0, 7] = jnp.where(pos, t2, zero)
    pay_ref[0, 8] = jnp.where(pos, t3, zero)

    fgf = fg.astype(jnp.float32)
    r0 = lax.broadcasted_iota(jnp.int32, (_LANES, _LANES), 0)
    c0 = lax.broadcasted_iota(jnp.int32, (_LANES, _LANES), 1)
    upper = (r0 <= c0).astype(jnp.float32)
    incl = jnp.dot(fgf, upper, preferred_element_type=jnp.float32)
    rowsum = incl[:, _LANES - 1 :]
    rr = lax.broadcasted_iota(jnp.int32, (_ROWS, _ROWS), 0)
    cc = lax.broadcasted_iota(jnp.int32, (_ROWS, _ROWS), 1)
    strict_lower = (cc < rr).astype(jnp.float32)
    offs = jnp.dot(strict_lower, rowsum, preferred_element_type=jnp.float32)
    pf_excl = incl + offs - fgf
    nfg = jnp.sum(fgf)
    lin = (
        lax.broadcasted_iota(jnp.int32, (_ROWS, _LANES), 0) * _LANES
        + lax.broadcasted_iota(jnp.int32, (_ROWS, _LANES), 1)
    ).astype(jnp.float32)
    destf = jnp.where(fg, pf_excl, nfg + lin - pf_excl)
    dest_ref[0] = destf.astype(jnp.int32)


_UNROLL = 8


def _scatter_all(nchunk, dest_v, in_v, out_v):
    def body(i, carry):
        off = i * (16 * _UNROLL)
        for u in range(_UNROLL):
            d = dest_v[pl.ds(off + u * 16, 16)]
            v = in_v[pl.ds(off + u * 16, 16)]
            plsc.store_scatter(out_v, [d], v)
        return carry

    lax.fori_loop(0, nchunk // _UNROLL, body, 0)


def _sc_body(
    B,
    nchunk,
    pay_hbm,
    dest_hbm,
    out_hbm,
    dest_v,
    in0_v,
    in1_v,
    out0_v,
    out1_v,
    sem0,
    sem1,
    sem2,
):
    info = plsc.get_sparse_core_info()
    nworker = info.num_cores * info.num_subcores
    wid = lax.axis_index("s") * info.num_cores + lax.axis_index("c")
    wpb = nworker // B
    b = wid // wpb
    lane = wid % wpb
    c0 = lane
    c1 = lane + wpb
    has2 = c1 < _NCOMP
    cp_d = pltpu.async_copy(dest_hbm.at[b], dest_v, sem2)
    cp0 = pltpu.async_copy(pay_hbm.at[b, c0], in0_v, sem0)

    @pl.when(has2)
    def _():
        pltpu.async_copy(pay_hbm.at[b, c1], in1_v, sem1)

    cp_d.wait()
    cp0.wait()
    _scatter_all(nchunk, dest_v, in0_v, out0_v)
    out_cp0 = pltpu.async_copy(out0_v, out_hbm.at[b, c0], sem0)

    @pl.when(has2)
    def _():
        pltpu.make_async_copy(pay_hbm.at[b, c1], in1_v, sem1).wait()
        _scatter_all(nchunk, dest_v, in1_v, out1_v)
        pltpu.sync_copy(out1_v, out_hbm.at[b, c1])

    out_cp0.wait()


def kernel(all_rois, gt_boxes):
    B, N, _ = all_rois.shape
    K = gt_boxes.shape[1]
    N2 = N + K
    pad = _N2P - N2
    z = jnp.zeros((B, pad), dtype=jnp.float32)
    x1 = jnp.concatenate([all_rois[:, :, 1], gt_boxes[:, :, 0], z], axis=1)
    y1 = jnp.concatenate([all_rois[:, :, 2], gt_boxes[:, :, 1], z], axis=1)
    x2 = jnp.concatenate([all_rois[:, :, 3], gt_boxes[:, :, 2], z], axis=1)
    y2 = jnp.concatenate([all_rois[:, :, 4], gt_boxes[:, :, 3], z], axis=1)
    shape3 = (B, _ROWS, _LANES)
    x1, y1, x2, y2 = (a.reshape(shape3) for a in (x1, y1, x2, y2))

    pay, dest = pl.pallas_call(
        functools.partial(_tc_body, K),
        grid=(B,),
        in_specs=[
            pl.BlockSpec((1, K, 5), lambda b: (b, 0, 0), memory_space=pltpu.SMEM),
            pl.BlockSpec((1, _ROWS, _LANES), lambda b: (b, 0, 0)),
            pl.BlockSpec((1, _ROWS, _LANES), lambda b: (b, 0, 0)),
            pl.BlockSpec((1, _ROWS, _LANES), lambda b: (b, 0, 0)),
            pl.BlockSpec((1, _ROWS, _LANES), lambda b: (b, 0, 0)),
        ],
        out_specs=[
            pl.BlockSpec((1, _NCOMP, _ROWS, _LANES), lambda b: (b, 0, 0, 0)),
            pl.BlockSpec((1, _ROWS, _LANES), lambda b: (b, 0, 0)),
        ],
        out_shape=[
            jax.ShapeDtypeStruct((B, _NCOMP, _ROWS, _LANES), jnp.float32),
            jax.ShapeDtypeStruct((B, _ROWS, _LANES), jnp.int32),
        ],
    )(gt_boxes, x1, y1, x2, y2)

    pay = pay.reshape(B, _NCOMP, _N2P)
    dest = dest.reshape(B, _N2P)

    mesh = plsc.VectorSubcoreMesh(core_axis_name="c", subcore_axis_name="s")
    out = pl.kernel(
        functools.partial(_sc_body, B, _N2P // 16),
        out_type=jax.ShapeDtypeStruct((B, _NCOMP, _N2P), jnp.float32),
        mesh=mesh,
        compiler_params=pltpu.CompilerParams(
            use_tc_tiling_on_sc=False, needs_layout_passes=False
        ),
        scratch_types=[
            pltpu.VMEM((_N2P,), jnp.int32),
            pltpu.VMEM((_N2P,), jnp.float32),
            pltpu.VMEM((_N2P,), jnp.float32),
            pltpu.VMEM((_N2P,), jnp.float32),
            pltpu.VMEM((_N2P,), jnp.float32),
            pltpu.SemaphoreType.DMA,
            pltpu.SemaphoreType.DMA,
            pltpu.SemaphoreType.DMA,
        ],
    )(pay, dest)

    o = out[:, :, :N2]
    labels_batch = o[:, 4]
    bcol = jnp.broadcast_to(
        jnp.arange(B, dtype=jnp.float32)[:, None, None], (B, N2, 1)
    )
    rois_batch = jnp.concatenate([bcol, jnp.moveaxis(o[:, 0:4], 1, 2)], axis=2)
    bbox_targets = jnp.moveaxis(o[:, 5:9], 1, 2)
    pos4 = jnp.broadcast_to((labels_batch > 0)[:, :, None], (B, N2, 4))
    bbox_inside_weights = pos4.astype(jnp.float32)
    bbox_outside_weights = pos4.astype(jnp.float32)
    return (
        rois_batch,
        labels_batch,
        bbox_targets,
        bbox_inside_weights,
        bbox_outside_weights,
    )
